# baseline (speedup 1.0000x reference)
import functools

import jax
import jax.numpy as jnp
from jax import lax
from jax.experimental import pallas as pl
from jax.experimental.pallas import tpu as pltpu
from jax.experimental.pallas import tpu_sc as plsc

NG = 2
NV = 320
DG = 128
KDIM = 1024
NTOK = 8 * 1024
NROW = NTOK * NG
SINK_ITERS = 20
TOK_STEP = 2048
NSTEPS = NTOK // TOK_STEP
CHL = 4096
NCH = NROW // CHL


def _tc_body(hs_ref, w_ref, b_ref, idx_ref, perp_ref, la_ref):
    i = pl.program_id(0)
    hs_blk = hs_ref[...]
    for g in range(NG):
        hb = jnp.dot(hs_blk, w_ref[g], preferred_element_type=jnp.float32)
        hb = jnp.exp(hb + b_ref[0, g])
        la_ref[:, pl.ds(g * NTOK + i * TOK_STEP, TOK_STEP)] = hb.T

    @pl.when(i == NSTEPS - 1)
    def _():
        def one_iter(_, w):
            def body(c, acc):
                blk = la_ref[:, pl.ds(c * CHL, CHL)]
                r = jnp.sum(blk * w, axis=0, keepdims=True)
                u = 1.0 / r
                return acc + jnp.sum(blk * u, axis=1, keepdims=True)

            acc = lax.fori_loop(
                0, NCH, body, jnp.zeros((NV, 1), jnp.float32), unroll=2)
            return (NROW / NV) / acc

        w = lax.fori_loop(0, SINK_ITERS, one_iter,
                          jnp.full((NV, 1), 1.0, jnp.float32), unroll=False)

        viota = lax.broadcasted_iota(jnp.int32, (NV, CHL), 0)

        def finish(c, carry):
            cnt0, cnt1 = carry
            blk = la_ref[:, pl.ds(c * CHL, CHL)] * w
            m = jnp.max(blk, axis=0, keepdims=True)
            idx = jnp.min(jnp.where(blk == m, viota, NV), axis=0, keepdims=True)
            is_g1 = c >= (NCH // 2)
            idx_ref[0, pl.ds(c * CHL, CHL)] = idx[0, :] + jnp.where(is_g1, NV, 0)
            onehot = (viota == idx).astype(jnp.float32)
            part = jnp.sum(onehot, axis=1, keepdims=True)
            cnt0 = cnt0 + jnp.where(is_g1, 0.0, part)
            cnt1 = cnt1 + jnp.where(is_g1, part, 0.0)
            return cnt0, cnt1

        z = jnp.zeros((NV, 1), jnp.float32)
        cnt0, cnt1 = lax.fori_loop(0, NCH, finish, (z, z), unroll=False)
        perp = jnp.zeros((1, 1), jnp.float32)
        for cnt in (cnt0, cnt1):
            avg = cnt * (1.0 / NTOK)
            perp += jnp.exp(
                -jnp.sum(avg * jnp.log(avg + 1e-7), axis=0, keepdims=True))
        perp_ref[...] = perp


def _tc_quantize(hs2d, w3, b3):
    return pl.pallas_call(
        _tc_body,
        grid=(NSTEPS,),
        in_specs=[
            pl.BlockSpec((TOK_STEP, KDIM), lambda i: (i, 0)),
            pl.BlockSpec((NG, KDIM, NV), lambda i: (0, 0, 0)),
            pl.BlockSpec((1, NG, NV), lambda i: (0, 0, 0)),
        ],
        out_specs=[
            pl.BlockSpec((1, NROW), lambda i: (0, 0)),
            pl.BlockSpec((1, 1), lambda i: (0, 0)),
        ],
        out_shape=[
            jax.ShapeDtypeStruct((1, NROW), jnp.int32),
            jax.ShapeDtypeStruct((1, 1), jnp.float32),
        ],
        scratch_shapes=[pltpu.VMEM((NV, NROW), jnp.float32)],
        compiler_params=pltpu.CompilerParams(
            dimension_semantics=("arbitrary",),
        ),
    )(hs2d, w3, b3)


def _make_sc_gather():
    info = plsc.get_sparse_core_info()
    nw = info.num_cores * info.num_subcores
    b_per_w = NROW // nw
    nidx = b_per_w // 128
    mesh = plsc.VectorSubcoreMesh(core_axis_name="c", subcore_axis_name="s")

    @functools.partial(
        pl.kernel,
        out_type=jax.ShapeDtypeStruct((NROW, DG), jnp.float32),
        mesh=mesh,
        scratch_types=[
            pltpu.VMEM((nidx, 128), jnp.int32),
            pltpu.VMEM((b_per_w, DG), jnp.float32),
            pltpu.SemaphoreType.DMA,
        ],
    )
    def sc_gather(table_hbm, idx_hbm, out_hbm, idx_v, rows_v, sem):
        wid = lax.axis_index("s") * info.num_cores + lax.axis_index("c")
        base = wid * b_per_w
        pltpu.sync_copy(idx_hbm.at[pl.ds(wid * nidx, nidx)], idx_v)
        copies = []
        for j in range(nidx):
            copies.append(pltpu.async_copy(
                table_hbm.at[idx_v.at[j]],
                rows_v.at[pl.ds(j * 128, 128)], sem))
        for cp in copies:
            cp.wait()
        pltpu.sync_copy(rows_v, out_hbm.at[pl.ds(base, b_per_w)])

    return sc_gather


_sc_gather = _make_sc_gather()


def kernel(hidden_states, W, bias, codevectors):
    b, l, _ = hidden_states.shape
    hs2d = hidden_states.reshape(b * l, KDIM)
    w3 = W.reshape(KDIM, NG, NV).transpose(1, 0, 2)
    b3 = bias.reshape(1, NG, NV)
    idx, perp = _tc_quantize(hs2d, w3, b3)
    idx_flat = idx.reshape(NG, NTOK).T.reshape(NROW // 128, 128)
    table = codevectors.reshape(NG * NV, DG)
    rows = _sc_gather(table, idx_flat)
    out = rows.reshape(b, l, NG * DG)
    return out, perp.reshape(())


# speedup vs baseline: 1.0413x; 1.0413x over previous
import functools

import jax
import jax.numpy as jnp
from jax import lax
from jax.experimental import pallas as pl
from jax.experimental.pallas import tpu as pltpu
from jax.experimental.pallas import tpu_sc as plsc

NG = 2
NV = 320
DG = 128
KDIM = 1024
NTOK = 8 * 1024
NROW = NTOK * NG
SINK_ITERS = 20
TOK_STEP = 2048
NSTEPS = NTOK // TOK_STEP
CHL = 2048
NCH = NROW // CHL


def _tc_body(hs_ref, w_ref, b_ref, idx_ref, perp_ref, la_ref):
    i = pl.program_id(0)
    hs_blk = hs_ref[...]
    for g in range(NG):
        hb = jnp.dot(hs_blk, w_ref[g], preferred_element_type=jnp.float32)
        hb = jnp.exp(hb + b_ref[0, g])
        la_ref[:, pl.ds(g * NTOK + i * TOK_STEP, TOK_STEP)] = hb.T

    @pl.when(i == NSTEPS - 1)
    def _():
        def one_iter(_, w):
            def body(c, acc):
                blk = la_ref[:, pl.ds(c * CHL, CHL)]
                r = jnp.sum(blk * w, axis=0, keepdims=True)
                u = 1.0 / r
                return acc + jnp.sum(blk * u, axis=1, keepdims=True)

            acc = lax.fori_loop(
                0, NCH, body, jnp.zeros((NV, 1), jnp.float32), unroll=8)
            return (NROW / NV) / acc

        w = lax.fori_loop(0, SINK_ITERS, one_iter,
                          jnp.full((NV, 1), 1.0, jnp.float32), unroll=False)

        viota = lax.broadcasted_iota(jnp.int32, (NV, CHL), 0)

        def finish(c, carry):
            cnt0, cnt1 = carry
            blk = la_ref[:, pl.ds(c * CHL, CHL)] * w
            m = jnp.max(blk, axis=0, keepdims=True)
            idx = jnp.min(jnp.where(blk == m, viota, NV), axis=0, keepdims=True)
            is_g1 = c >= (NCH // 2)
            idx_ref[0, pl.ds(c * CHL, CHL)] = idx[0, :] + jnp.where(is_g1, NV, 0)
            onehot = (viota == idx).astype(jnp.float32)
            part = jnp.sum(onehot, axis=1, keepdims=True)
            cnt0 = cnt0 + jnp.where(is_g1, 0.0, part)
            cnt1 = cnt1 + jnp.where(is_g1, part, 0.0)
            return cnt0, cnt1

        z = jnp.zeros((NV, 1), jnp.float32)
        cnt0, cnt1 = lax.fori_loop(0, NCH, finish, (z, z), unroll=False)
        perp = jnp.zeros((1, 1), jnp.float32)
        for cnt in (cnt0, cnt1):
            avg = cnt * (1.0 / NTOK)
            perp += jnp.exp(
                -jnp.sum(avg * jnp.log(avg + 1e-7), axis=0, keepdims=True))
        perp_ref[...] = perp


def _tc_quantize(hs2d, w3, b3):
    return pl.pallas_call(
        _tc_body,
        grid=(NSTEPS,),
        in_specs=[
            pl.BlockSpec((TOK_STEP, KDIM), lambda i: (i, 0)),
            pl.BlockSpec((NG, KDIM, NV), lambda i: (0, 0, 0)),
            pl.BlockSpec((1, NG, NV), lambda i: (0, 0, 0)),
        ],
        out_specs=[
            pl.BlockSpec((1, NROW), lambda i: (0, 0)),
            pl.BlockSpec((1, 1), lambda i: (0, 0)),
        ],
        out_shape=[
            jax.ShapeDtypeStruct((1, NROW), jnp.int32),
            jax.ShapeDtypeStruct((1, 1), jnp.float32),
        ],
        scratch_shapes=[pltpu.VMEM((NV, NROW), jnp.float32)],
        compiler_params=pltpu.CompilerParams(
            dimension_semantics=("arbitrary",),
        ),
    )(hs2d, w3, b3)


def _make_sc_gather():
    info = plsc.get_sparse_core_info()
    nw = info.num_cores * info.num_subcores
    b_per_w = NROW // nw
    nidx = b_per_w // 128
    mesh = plsc.VectorSubcoreMesh(core_axis_name="c", subcore_axis_name="s")

    @functools.partial(
        pl.kernel,
        out_type=jax.ShapeDtypeStruct((NROW, DG), jnp.float32),
        mesh=mesh,
        scratch_types=[
            pltpu.VMEM((nidx, 128), jnp.int32),
            pltpu.VMEM((b_per_w, DG), jnp.float32),
            pltpu.SemaphoreType.DMA,
        ],
    )
    def sc_gather(table_hbm, idx_hbm, out_hbm, idx_v, rows_v, sem):
        wid = lax.axis_index("s") * info.num_cores + lax.axis_index("c")
        base = wid * b_per_w
        pltpu.sync_copy(idx_hbm.at[pl.ds(wid * nidx, nidx)], idx_v)
        copies = []
        for j in range(nidx):
            copies.append(pltpu.async_copy(
                table_hbm.at[idx_v.at[j]],
                rows_v.at[pl.ds(j * 128, 128)], sem))
        for cp in copies:
            cp.wait()
        pltpu.sync_copy(rows_v, out_hbm.at[pl.ds(base, b_per_w)])

    return sc_gather


_sc_gather = _make_sc_gather()


def kernel(hidden_states, W, bias, codevectors):
    b, l, _ = hidden_states.shape
    hs2d = hidden_states.reshape(b * l, KDIM)
    w3 = W.reshape(KDIM, NG, NV).transpose(1, 0, 2)
    b3 = bias.reshape(1, NG, NV)
    idx, perp = _tc_quantize(hs2d, w3, b3)
    idx_flat = idx.reshape(NG, NTOK).T.reshape(NROW // 128, 128)
    table = codevectors.reshape(NG * NV, DG)
    rows = _sc_gather(table, idx_flat)
    out = rows.reshape(b, l, NG * DG)
    return out, perp.reshape(())
